# baseline (speedup 1.0000x reference)
import functools

import jax
import jax.numpy as jnp
from jax import lax
from jax.experimental import pallas as pl
from jax.experimental.pallas import tpu as pltpu
from jax.experimental.pallas import tpu_sc as plsc

NT = 64
WIN = 16
LANES = 16

_B, _N, _D = 32, 1024, 768
_ROWS_IN = _B * _N
_ROWS_OUT = _B * NT
_NWORKERS = 32
_W_IN = _ROWS_IN // _NWORKERS
_W_OUT = _ROWS_OUT // _NWORKERS
_CH_OUT = 4
_CH_IN = _CH_OUT * WIN
_NCH = _W_OUT // _CH_OUT
_NBUF = 2


def _sc_body(x_hbm, o_hbm, in_buf, out_buf, in_sems, out_sems):
    c = lax.axis_index("c")
    s = lax.axis_index("s")
    wid = s * 2 + c
    in_base = wid * _W_IN
    out_base = wid * _W_OUT

    def start_in(g, slot):
        pltpu.make_async_copy(
            x_hbm.at[pl.ds(in_base + g * _CH_IN, _CH_IN)],
            in_buf.at[slot],
            in_sems.at[slot],
        ).start()

    def wait_in(slot):
        pltpu.make_async_copy(
            x_hbm.at[pl.ds(0, _CH_IN)],
            in_buf.at[slot],
            in_sems.at[slot],
        ).wait()

    def start_out(g, slot):
        pltpu.make_async_copy(
            out_buf.at[slot],
            o_hbm.at[pl.ds(out_base + g * _CH_OUT, _CH_OUT)],
            out_sems.at[slot],
        ).start()

    def wait_out(g, slot):
        pltpu.make_async_copy(
            out_buf.at[slot],
            o_hbm.at[pl.ds(out_base + g * _CH_OUT, _CH_OUT)],
            out_sems.at[slot],
        ).wait()

    for b in range(_NBUF):
        start_in(b, b)

    scale = jnp.float32(1.0 / WIN)

    def chunk_group(g0):
        for b in range(_NBUF):
            g = g0 + b
            wait_in(b)

            @pl.when(g >= _NBUF)
            def _():
                wait_out(g - _NBUF, b)

            ncol = _D // LANES
            for o in range(_CH_OUT):
                def row_body(r, accs, o=o, b=b):
                    row = o * WIN + r
                    return tuple(
                        accs[j] + in_buf[b, row, pl.ds(j * LANES, LANES)]
                        for j in range(ncol)
                    )

                accs0 = tuple(
                    in_buf[b, o * WIN, pl.ds(j * LANES, LANES)]
                    for j in range(ncol)
                )
                accs = lax.fori_loop(1, WIN, row_body, accs0)
                for j in range(ncol):
                    out_buf[b, o, pl.ds(j * LANES, LANES)] = accs[j] * scale

            start_out(g, b)

            @pl.when(g + _NBUF < _NCH)
            def _():
                start_in(g + _NBUF, b)

    pl.loop(0, _NCH, step=_NBUF)(chunk_group)

    for b in range(_NBUF):
        wait_out(_NCH - _NBUF + b, b)


@functools.partial(
    pl.kernel,
    out_type=jax.ShapeDtypeStruct((_ROWS_OUT, _D), jnp.float32),
    mesh=plsc.VectorSubcoreMesh(core_axis_name="c", subcore_axis_name="s"),
    scratch_types=[
        pltpu.VMEM((_NBUF, _CH_IN, _D), jnp.float32),
        pltpu.VMEM((_NBUF, _CH_OUT, _D), jnp.float32),
        pltpu.SemaphoreType.DMA((_NBUF,)),
        pltpu.SemaphoreType.DMA((_NBUF,)),
    ],
)
def _sc_pool(x_hbm, o_hbm, in_buf, out_buf, in_sems, out_sems):
    _sc_body(x_hbm, o_hbm, in_buf, out_buf, in_sems, out_sems)


def kernel(features):
    B, N, D = features.shape
    x = features.reshape(B * N, D)
    out = _sc_pool(x)
    return out.reshape(B, NT, D)


# speedup vs baseline: 1.1789x; 1.1789x over previous
import functools

import jax
import jax.numpy as jnp
from jax import lax
from jax.experimental import pallas as pl
from jax.experimental.pallas import tpu as pltpu
from jax.experimental.pallas import tpu_sc as plsc

NT = 64
WIN = 16
LANES = 16

_B, _N, _D = 32, 1024, 768
_SC_B = 8
_NWORKERS = 32
_NBUF = 2
_CH_OUT = 4
_CH_IN = _CH_OUT * WIN


def _make_sc_pool(sc_b):
    rows_out = sc_b * NT
    w_out = rows_out // _NWORKERS
    w_in = w_out * WIN
    nch = w_out // _CH_OUT
    ncol = _D // LANES
    scale = jnp.float32(1.0 / WIN)

    def body(x_hbm, o_hbm, in_buf, out_buf, in_sems, out_sems):
        c = lax.axis_index("c")
        s = lax.axis_index("s")
        wid = s * 2 + c
        in_base = wid * w_in
        out_base = wid * w_out

        def start_in(g, slot):
            pltpu.make_async_copy(
                x_hbm.at[pl.ds(in_base + g * _CH_IN, _CH_IN)],
                in_buf.at[slot],
                in_sems.at[slot],
            ).start()

        def wait_in(slot):
            pltpu.make_async_copy(
                x_hbm.at[pl.ds(0, _CH_IN)],
                in_buf.at[slot],
                in_sems.at[slot],
            ).wait()

        def start_out(g, slot):
            pltpu.make_async_copy(
                out_buf.at[slot],
                o_hbm.at[pl.ds(out_base + g * _CH_OUT, _CH_OUT)],
                out_sems.at[slot],
            ).start()

        def wait_out(g, slot):
            pltpu.make_async_copy(
                out_buf.at[slot],
                o_hbm.at[pl.ds(out_base + g * _CH_OUT, _CH_OUT)],
                out_sems.at[slot],
            ).wait()

        for b in range(_NBUF):
            start_in(b, b)

        def chunk_group(g0):
            for b in range(_NBUF):
                g = g0 + b
                wait_in(b)

                @pl.when(g >= _NBUF)
                def _():
                    wait_out(g - _NBUF, b)

                for o in range(_CH_OUT):
                    def row_body(r, accs, o=o, b=b):
                        row = o * WIN + r
                        return tuple(
                            accs[j] + in_buf[b, row, pl.ds(j * LANES, LANES)]
                            for j in range(ncol)
                        )

                    accs0 = tuple(
                        in_buf[b, o * WIN, pl.ds(j * LANES, LANES)]
                        for j in range(ncol)
                    )
                    accs = lax.fori_loop(1, WIN, row_body, accs0)
                    for j in range(ncol):
                        out_buf[b, o, pl.ds(j * LANES, LANES)] = accs[j] * scale

                start_out(g, b)

                @pl.when(g + _NBUF < nch)
                def _():
                    start_in(g + _NBUF, b)

        pl.loop(0, nch, step=_NBUF)(chunk_group)

        for b in range(_NBUF):
            wait_out(nch - _NBUF + b, b)

    return pl.kernel(
        body,
        out_type=jax.ShapeDtypeStruct((rows_out, _D), jnp.float32),
        mesh=plsc.VectorSubcoreMesh(core_axis_name="c", subcore_axis_name="s"),
        scratch_types=[
            pltpu.VMEM((_NBUF, _CH_IN, _D), jnp.float32),
            pltpu.VMEM((_NBUF, _CH_OUT, _D), jnp.float32),
            pltpu.SemaphoreType.DMA((_NBUF,)),
            pltpu.SemaphoreType.DMA((_NBUF,)),
        ],
    )


def _tc_body(x_ref, o_ref):
    o_ref[0, :, :] = jnp.sum(x_ref[0], axis=1) * (1.0 / WIN)


def kernel(features):
    B, N, D = features.shape
    x_flat = features.reshape(B * N, D)
    out_sc = _make_sc_pool(_SC_B)(x_flat)

    tc_b = B - _SC_B
    x4 = features.reshape(B, NT, WIN, D)
    out_tc = pl.pallas_call(
        _tc_body,
        grid=(tc_b,),
        in_specs=[
            pl.BlockSpec((1, NT, WIN, D), lambda b: (b + _SC_B, 0, 0, 0))
        ],
        out_specs=pl.BlockSpec((1, NT, D), lambda b: (b, 0, 0)),
        out_shape=jax.ShapeDtypeStruct((tc_b, NT, D), jnp.float32),
    )(x4)

    return jnp.concatenate([out_sc.reshape(_SC_B, NT, D), out_tc], axis=0)


# speedup vs baseline: 1.6814x; 1.4263x over previous
import jax
import jax.numpy as jnp
from jax.experimental import pallas as pl

NT = 64


def _pool_body(x_ref, o_ref):
    win = x_ref.shape[2]
    o_ref[0, :, :] = jnp.sum(x_ref[0], axis=1) * (1.0 / win)


def kernel(features):
    B, N, D = features.shape
    win = N // NT
    x = features.reshape(B, NT, win, D)
    out = pl.pallas_call(
        _pool_body,
        grid=(B,),
        in_specs=[pl.BlockSpec((1, NT, win, D), lambda b: (b, 0, 0, 0))],
        out_specs=pl.BlockSpec((1, NT, D), lambda b: (b, 0, 0)),
        out_shape=jax.ShapeDtypeStruct((B, NT, D), jnp.float32),
    )(x)
    return out
